# baseline (speedup 1.0000x reference)
import jax
import jax.numpy as jnp
from jax.experimental import pallas as pl

B, T, V, D = 4096, 8, 1024, 256
BB = 256
NCHUNK = V // 128


def _decode_block(msg_ref, cb_ref, out_ref):
    m4 = msg_ref[...].reshape(BB, T, NCHUNK, 128)
    best = m4[:, :, 0, :]
    bidx = jnp.zeros((BB, T, 128), jnp.int32)
    for c in range(1, NCHUNK):
        cur = m4[:, :, c, :]
        better = cur > best
        best = jnp.where(better, cur, best)
        bidx = jnp.where(better, c, bidx)
    lane = jax.lax.broadcasted_iota(jnp.int32, (BB, T, 128), 2)
    vfull = bidx * 128 + lane
    mx = jnp.max(best, axis=-1, keepdims=True)
    code = jnp.min(jnp.where(best == mx, vfull, V), axis=-1)
    iota3 = jax.lax.broadcasted_iota(jnp.int32, (BB, T, V), 2)
    oh = (iota3 == code[:, :, None]).astype(jnp.bfloat16)
    acc = jnp.zeros((BB, D), jnp.float32)
    for t in range(T):
        g = jax.lax.dot(oh[:, t, :], cb_ref[t], preferred_element_type=jnp.float32)
        acc = acc + g
        norm = jnp.sqrt(jnp.sum(acc * acc, axis=-1, keepdims=True))
        out_ref[:, t, :] = acc * (1.0 / jnp.maximum(norm, 1e-12))


@jax.jit
def kernel(message, codebooks):
    cb16 = codebooks.astype(jnp.bfloat16)
    return pl.pallas_call(
        _decode_block,
        grid=(B // BB,),
        in_specs=[
            pl.BlockSpec((BB, T, V), lambda i: (i, 0, 0)),
            pl.BlockSpec((T, V, D), lambda i: (0, 0, 0)),
        ],
        out_specs=pl.BlockSpec((BB, T, D), lambda i: (i, 0, 0)),
        out_shape=jax.ShapeDtypeStruct((B, T, D), jnp.float32),
    )(message, cb16)


# speedup vs baseline: 1.5658x; 1.5658x over previous
import jax
import jax.numpy as jnp
from jax.experimental import pallas as pl

B, T, V, D = 4096, 8, 1024, 256
BB = 256
DE = D + 128


def _decode_block(msg_ref, cb_ref, out_ref):
    acc = jnp.zeros((BB, D), jnp.float32)
    cnt = jnp.zeros((BB, 128), jnp.float32)
    for t in range(T):
        mt = msg_ref[:, t * V : (t + 1) * V]
        mx = jnp.max(mt, axis=-1, keepdims=True)
        oh = (mt == mx).astype(jnp.bfloat16)
        ge = jax.lax.dot(oh, cb_ref[t], preferred_element_type=jnp.float32)
        acc = acc + ge[:, :D]
        cnt = cnt + ge[:, D:]
        norm = jnp.sqrt(jnp.sum(acc * acc, axis=-1, keepdims=True))
        out_ref[:, t * D : (t + 1) * D] = acc * (1.0 / jnp.maximum(norm, 1e-12))

    bad = jnp.max(cnt) > T + 0.5

    @pl.when(bad)
    def _fixup():
        acc2 = jnp.zeros((BB, D), jnp.float32)
        for t in range(T):
            mt = msg_ref[:, t * V : (t + 1) * V]
            mx = jnp.max(mt, axis=-1, keepdims=True)
            iota2 = jax.lax.broadcasted_iota(jnp.int32, (BB, V), 1)
            code = jnp.min(jnp.where(mt == mx, iota2, V), axis=-1, keepdims=True)
            oh2 = (iota2 == code).astype(jnp.bfloat16)
            g2 = jax.lax.dot(oh2, cb_ref[t], preferred_element_type=jnp.float32)
            acc2 = acc2 + g2[:, :D]
            norm = jnp.sqrt(jnp.sum(acc2 * acc2, axis=-1, keepdims=True))
            out_ref[:, t * D : (t + 1) * D] = acc2 * (1.0 / jnp.maximum(norm, 1e-12))


@jax.jit
def kernel(message, codebooks):
    cb16 = codebooks.astype(jnp.bfloat16)
    cb_ext = jnp.concatenate(
        [
            cb16,
            jnp.ones((T, V, 1), jnp.bfloat16),
            jnp.zeros((T, V, 127), jnp.bfloat16),
        ],
        axis=-1,
    )
    out = pl.pallas_call(
        _decode_block,
        grid=(B // BB,),
        in_specs=[
            pl.BlockSpec((BB, T * V), lambda i: (i, 0)),
            pl.BlockSpec((T, V, DE), lambda i: (0, 0, 0)),
        ],
        out_specs=pl.BlockSpec((BB, T * D), lambda i: (i, 0)),
        out_shape=jax.ShapeDtypeStruct((B, T * D), jnp.float32),
    )(message.reshape(B, T * V), cb_ext)
    return out.reshape(B, T, D)


# speedup vs baseline: 3.9126x; 2.4987x over previous
import jax
import jax.numpy as jnp
from jax.experimental import pallas as pl

B, T, V, D = 4096, 8, 1024, 256
BB = 512


def _decode_block(msg_ref, cb_ref, out_ref):
    m = msg_ref[...]
    mx = jnp.max(m, axis=-1, keepdims=True)
    iota3 = jax.lax.broadcasted_iota(jnp.int32, (BB, T, V), 2)
    codes = jnp.min(jnp.where(m == mx, iota3, V), axis=-1)
    iota2 = jax.lax.broadcasted_iota(jnp.int32, (BB, V), 1)
    acc = jnp.zeros((BB, D), jnp.float32)
    for t in range(T):
        onehot = (iota2 == codes[:, t : t + 1]).astype(jnp.bfloat16)
        g = jax.lax.dot(onehot, cb_ref[t], preferred_element_type=jnp.float32)
        acc = acc + g
        norm = jnp.sqrt(jnp.sum(acc * acc, axis=-1, keepdims=True))
        out_ref[:, t, :] = acc * (1.0 / jnp.maximum(norm, 1e-12))


@jax.jit
def kernel(message, codebooks):
    cb16 = codebooks.astype(jnp.bfloat16)
    return pl.pallas_call(
        _decode_block,
        grid=(B // BB,),
        in_specs=[
            pl.BlockSpec((BB, T, V), lambda i: (i, 0, 0)),
            pl.BlockSpec((T, V, D), lambda i: (0, 0, 0)),
        ],
        out_specs=pl.BlockSpec((BB, T, D), lambda i: (i, 0, 0)),
        out_shape=jax.ShapeDtypeStruct((B, T, D), jnp.float32),
    )(message, cb16)
